# baseline (speedup 1.0000x reference)
import jax
import jax.numpy as jnp
from jax.experimental import pallas as pl
from jax.experimental.pallas import tpu as pltpu

_N = 33554432
_C = 2048
_R = _N // _C
_BR = 1024
_G = _R // _BR


def _softmax_body(x_ref, o_ref, acc_ref):
    p = pl.program_id(0)
    i = pl.program_id(1)

    @pl.when((p == 0) & (i == 0))
    def _init():
        acc_ref[...] = jnp.zeros_like(acc_ref)

    @pl.when(p == 0)
    def _accumulate():
        acc_ref[...] += jnp.sum(jnp.exp(x_ref[...]), axis=0, keepdims=True)

    @pl.when(p == 1)
    def _scale():
        total = jnp.sum(acc_ref[...])
        o_ref[...] = jnp.exp(x_ref[...]) * (1.0 / total)


def kernel(x):
    x2 = x.reshape(_R, _C)
    out = pl.pallas_call(
        _softmax_body,
        out_shape=jax.ShapeDtypeStruct((_R, _C), jnp.float32),
        grid=(2, _G),
        in_specs=[pl.BlockSpec((_BR, _C), lambda p, i: (i, 0))],
        out_specs=pl.BlockSpec((_BR, _C), lambda p, i: (i * p, 0)),
        scratch_shapes=[pltpu.VMEM((1, _C), jnp.float32)],
        compiler_params=pltpu.CompilerParams(
            dimension_semantics=("arbitrary", "arbitrary"),
            vmem_limit_bytes=48 * 1024 * 1024,
        ),
        name="flat_softmax",
    )(x2)
    return out.reshape(_N)


# speedup vs baseline: 1.0062x; 1.0062x over previous
import jax
import jax.numpy as jnp
from jax.experimental import pallas as pl
from jax.experimental.pallas import tpu as pltpu

_N = 33554432
_BN = 1 << 21
_G = _N // _BN


def _softmax_body(x_ref, o_ref, acc_ref):
    p = pl.program_id(0)
    i = pl.program_id(1)

    @pl.when((p == 0) & (i == 0))
    def _init():
        acc_ref[0] = 0.0

    @pl.when(p == 0)
    def _accumulate():
        acc_ref[0] += jnp.sum(jnp.exp(x_ref[...]))

    @pl.when(p == 1)
    def _scale():
        o_ref[...] = jnp.exp(x_ref[...]) * (1.0 / acc_ref[0])


def kernel(x):
    return pl.pallas_call(
        _softmax_body,
        out_shape=jax.ShapeDtypeStruct((_N,), jnp.float32),
        grid=(2, _G),
        in_specs=[pl.BlockSpec((_BN,), lambda p, i: (i,))],
        out_specs=pl.BlockSpec((_BN,), lambda p, i: (i * p,)),
        scratch_shapes=[pltpu.SMEM((1,), jnp.float32)],
        compiler_params=pltpu.CompilerParams(
            dimension_semantics=("arbitrary", "arbitrary"),
            vmem_limit_bytes=48 * 1024 * 1024,
        ),
        name="flat_softmax",
    )(x)


# speedup vs baseline: 2.9609x; 2.9425x over previous
import jax
import jax.numpy as jnp
from jax.experimental import pallas as pl
from jax.experimental.pallas import tpu as pltpu

_N = 33554432
_BN = 1 << 21
_G = _N // _BN
_CH = 1 << 16
_K = _BN // _CH


def _softmax_body(x_ref, o_ref, acc_ref, inv_ref):
    p = pl.program_id(0)
    i = pl.program_id(1)

    @pl.when((p == 0) & (i == 0))
    def _init():
        acc_ref[...] = jnp.zeros_like(acc_ref)

    @pl.when(p == 0)
    def _accumulate():
        for k in range(_K):
            acc_ref[...] += jnp.exp(x_ref[pl.ds(k * _CH, _CH)])

    @pl.when((p == 1) & (i == 0))
    def _finalize():
        inv_ref[0] = 1.0 / jnp.sum(acc_ref[...])

    @pl.when(p == 1)
    def _scale():
        o_ref[...] = jnp.exp(x_ref[...]) * inv_ref[0]


def kernel(x):
    return pl.pallas_call(
        _softmax_body,
        out_shape=jax.ShapeDtypeStruct((_N,), jnp.float32),
        grid=(2, _G),
        in_specs=[pl.BlockSpec((_BN,), lambda p, i: (i,))],
        out_specs=pl.BlockSpec((_BN,), lambda p, i: (i * p,)),
        scratch_shapes=[
            pltpu.VMEM((_CH,), jnp.float32),
            pltpu.SMEM((1,), jnp.float32),
        ],
        compiler_params=pltpu.CompilerParams(
            dimension_semantics=("arbitrary", "arbitrary"),
            vmem_limit_bytes=48 * 1024 * 1024,
        ),
        name="flat_softmax",
    )(x)
